# baseline (speedup 1.0000x reference)
import functools

import jax
import jax.numpy as jnp
from jax import lax
from jax.experimental import pallas as pl
from jax.experimental.pallas import tpu as pltpu
from jax.experimental.pallas import tpu_sc as plsc

D = 64
NC = 2
NS = 16
NW = NC * NS
CH = 128
GPS = 4
SLAB = GPS * CH
PAIR = 2 * SLAB


@functools.partial(jax.jit, static_argnames=("m",))
def _two_gathers(idx1, idx2, table, m):
    n = idx1.size
    mesh = plsc.VectorSubcoreMesh(
        core_axis_name="c", subcore_axis_name="s", num_cores=NC, num_subcores=NS
    )

    def body(idx1_h, idx2_h, table_h, out1_h, out2_h,
             idx_a, idx_b, rows_a, rows_b, gs_a, gs_b, os_a, os_b):
        w = lax.axis_index("s") * NC + lax.axis_index("c")
        idx_bufs = (idx_a, idx_b)
        row_bufs = (rows_a, rows_b)
        gsems = (gs_a, gs_b)
        osems = (os_a, os_b)

        for idx_h, out_h in ((idx1_h, out1_h), (idx2_h, out2_h)):

            def stage_idx(p, ib, idx_h=idx_h):
                pltpu.sync_copy(idx_h.at[pl.ds((w * m + p) * 8, 8)],
                                idx_bufs[ib])

            def fire_g(ib, b):
                for j in range(GPS):
                    pltpu.async_copy(
                        table_h.at[idx_bufs[ib].at[b * GPS + j]],
                        row_bufs[b].at[pl.ds(j * CH, CH)], gsems[b])

            def drain_g(b):
                pltpu.make_async_copy(table_h.at[pl.ds(0, SLAB)],
                                      row_bufs[b], gsems[b]).wait()

            def fire_wb(sl, b, out_h=out_h):
                pltpu.async_copy(row_bufs[b],
                                 out_h.at[pl.ds((w * 2 * m + sl) * SLAB, SLAB)],
                                 osems[b])

            def drain_wb(b, out_h=out_h):
                pltpu.make_async_copy(row_bufs[b], out_h.at[pl.ds(0, SLAB)],
                                      osems[b]).wait()

            stage_idx(0, 0)
            stage_idx(1, 1)
            fire_g(0, 0)
            fire_g(0, 1)

            @pl.loop(0, m - 1, step=2)
            def _(pp):
                for q in range(2):
                    p = pp + q
                    drain_g(0)
                    fire_wb(2 * p, 0)
                    drain_g(1)
                    fire_wb(2 * p + 1, 1)
                    drain_wb(0)
                    fire_g(1 - q, 0)
                    drain_wb(1)
                    fire_g(1 - q, 1)
                    stage_idx(jnp.minimum(p + 2, m - 1), q)

            drain_g(0)
            fire_wb(2 * (m - 1), 0)
            drain_g(1)
            fire_wb(2 * m - 1, 1)
            drain_wb(0)
            drain_wb(1)

    call = pl.kernel(
        body,
        out_type=(jax.ShapeDtypeStruct((n, D), jnp.float32),
                  jax.ShapeDtypeStruct((n, D), jnp.float32)),
        mesh=mesh,
        scratch_types=(
            pltpu.VMEM((8, CH), jnp.int32),
            pltpu.VMEM((8, CH), jnp.int32),
            pltpu.VMEM((SLAB, D), jnp.float32),
            pltpu.VMEM((SLAB, D), jnp.float32),
            pltpu.SemaphoreType.DMA,
            pltpu.SemaphoreType.DMA,
            pltpu.SemaphoreType.DMA,
            pltpu.SemaphoreType.DMA,
        ),
        compiler_params=pltpu.CompilerParams(use_tc_tiling_on_sc=False),
    )
    return call(idx1, idx2, table)


def kernel(text, text2, table):
    b, l = text.shape
    n = b * l
    assert n % (NW * PAIR) == 0
    m = n // (NW * PAIR)
    assert m % 2 == 1
    idx1 = text.reshape(n // CH, CH)
    idx2 = text2.reshape(n // CH, CH)
    out1, out2 = _two_gathers(idx1, idx2, table, m)
    return out1.reshape(b, l, D), out2.reshape(b, l, D)


# speedup vs baseline: 1.4338x; 1.4338x over previous
import functools

import jax
import jax.numpy as jnp
from jax import lax
from jax.experimental import pallas as pl
from jax.experimental.pallas import tpu as pltpu
from jax.experimental.pallas import tpu_sc as plsc

D = 64
NC = 2
NS = 16
NW = NC * NS
CH = 128
GPS = 5
SLAB = GPS * CH


@functools.partial(jax.jit, static_argnames=("s_per",))
def _two_gathers(idx1, idx2, table, s_per):
    n = idx1.size
    mesh = plsc.VectorSubcoreMesh(
        core_axis_name="c", subcore_axis_name="s", num_cores=NC, num_subcores=NS
    )

    def body(idx1_h, idx2_h, table_h, out1_h, out2_h,
             idx_a, idx_b, rows_a, rows_b, gs_a, gs_b, os_a, os_b):
        w = lax.axis_index("s") * NC + lax.axis_index("c")
        idx_bufs = (idx_a, idx_b)
        row_bufs = (rows_a, rows_b)
        gsems = (gs_a, gs_b)
        osems = (os_a, os_b)

        for idx_h, out_h in ((idx1_h, out1_h), (idx2_h, out2_h)):

            def stage_idx(s, b, idx_h=idx_h):
                pltpu.sync_copy(idx_h.at[pl.ds((w * s_per + s) * SLAB, SLAB)],
                                idx_bufs[b])

            def fire_g(b):
                for j in range(GPS):
                    pltpu.async_copy(
                        table_h.at[idx_bufs[b].at[pl.ds(j * CH, CH)]],
                        row_bufs[b].at[pl.ds(j * CH, CH)], gsems[b])

            def drain_g(b):
                pltpu.make_async_copy(table_h.at[pl.ds(0, SLAB)],
                                      row_bufs[b], gsems[b]).wait()

            def fire_wb(s, b, out_h=out_h):
                pltpu.async_copy(
                    row_bufs[b],
                    out_h.at[pl.ds((w * s_per + s) * SLAB, SLAB), pl.ds(0, D)],
                    osems[b])

            def drain_wb(b, out_h=out_h):
                pltpu.make_async_copy(
                    row_bufs[b], out_h.at[pl.ds(0, SLAB), pl.ds(0, D)],
                    osems[b]).wait()

            stage_idx(0, 0)
            stage_idx(1, 1)
            fire_g(0)
            fire_g(1)

            @pl.loop(0, s_per - 2, step=2)
            def _(si):
                for b in range(2):
                    s = si + b
                    drain_g(b)
                    fire_wb(s, b)
                    stage_idx(s + 2, b)
                    drain_wb(b)
                    fire_g(b)

            for b in range(2):
                drain_g(b)
                fire_wb(s_per - 2 + b, b)
                drain_wb(b)

    call = pl.kernel(
        body,
        out_type=(jax.ShapeDtypeStruct((n, 2 * D), jnp.float32),
                  jax.ShapeDtypeStruct((n, 2 * D), jnp.float32)),
        mesh=mesh,
        scratch_types=(
            pltpu.VMEM((SLAB,), jnp.int32),
            pltpu.VMEM((SLAB,), jnp.int32),
            pltpu.VMEM((SLAB, D), jnp.float32),
            pltpu.VMEM((SLAB, D), jnp.float32),
            pltpu.SemaphoreType.DMA,
            pltpu.SemaphoreType.DMA,
            pltpu.SemaphoreType.DMA,
            pltpu.SemaphoreType.DMA,
        ),
        compiler_params=pltpu.CompilerParams(use_tc_tiling_on_sc=False),
    )
    return call(idx1, idx2, table)


def kernel(text, text2, table):
    b, l = text.shape
    n = b * l
    assert n % (NW * SLAB) == 0
    s_per = n // (NW * SLAB)
    assert s_per % 2 == 0 and s_per >= 4
    idx1 = text.reshape(n)
    idx2 = text2.reshape(n)
    out1, out2 = _two_gathers(idx1, idx2, table, s_per)
    return (out1[:, :D].reshape(b, l, D), out2[:, :D].reshape(b, l, D))
